# baseline (speedup 1.0000x reference)
import functools
import numpy as np
import jax
import jax.numpy as jnp
from jax import lax
from jax.experimental import pallas as pl
from jax.experimental.pallas import tpu as pltpu

_NUM_CLASSES = 80
_IMG_SIZE = 416.0
_FM = [13, 26, 52]
_LAST = [0, 507, 2535]
_TOTAL = 10647
_ANCH = [
    [(3.625, 2.8125), (4.875, 6.1875), (11.65625, 10.1875)],
    [(1.875, 3.8125), (3.875, 2.8125), (3.6875, 7.4375)],
    [(1.25, 1.625), (2.0, 3.75), (4.125, 2.875)],
]
_BS = 32
_NB = 20
_NTOT = float(_BS * _TOTAL)
_RPW = 9
_WROWS = _TOTAL // _RPW
_WCOLS = _RPW * 85


def _loss_body(pred_ref, tgt_ref, out_ref, noobj_ref, acc_ref):
    b = pl.program_id(0)

    @pl.when(b == 0)
    def _():
        for i in range(8):
            acc_ref[i] = 0.0

    noobj_ref[:, :] = jnp.ones((_TOTAL, 1), jnp.float32)

    idxs, keeps, txs, tys, tws, ths, wwhs, cis = [], [], [], [], [], [], [], []
    for n in range(_NB):
        t0 = tgt_ref[0, n, 0]
        t1 = tgt_ref[0, n, 1]
        t2 = tgt_ref[0, n, 2]
        t3 = tgt_ref[0, n, 3]
        t4 = tgt_ref[0, n, 4]
        keep = (t0 + t1 + t2 + t3 + t4) != 0.0

        bms, bas, bases = [], [], []
        for m in range(3):
            fs = float(_FM[m])
            gi = (t1 * fs).astype(jnp.int32)
            gj = (t2 * fs).astype(jnp.int32)
            gw = t3 * fs
            gh = t4 * fs
            ious = []
            for (aw, ah) in _ANCH[m]:
                inter = jnp.minimum(gw, aw) * jnp.minimum(gh, ah)
                union = gw * gh + aw * ah - inter + 1e-16
                ious.append(inter / union)
            bm = ious[0]
            ba = jnp.int32(0)
            ba = jnp.where(ious[1] > bm, 1, ba)
            bm = jnp.maximum(bm, ious[1])
            ba = jnp.where(ious[2] > bm, 2, ba)
            bm = jnp.maximum(bm, ious[2])
            base = _LAST[m] + 3 * gi * gj
            bases.append(base)
            bms.append(bm)
            bas.append(ba)
            for a in range(3):
                cond = keep & (ious[a] > 0.5)

                @pl.when(cond)
                def _(base=base, a=a):
                    noobj_ref[pl.ds(base + a, 1), :] = jnp.zeros((1, 1), jnp.float32)

        bf_bm = bms[0]
        base_sel = bases[0]
        ba_sel = bas[0]
        for m in (1, 2):
            better = bms[m] > bf_bm
            base_sel = jnp.where(better, bases[m], base_sel)
            ba_sel = jnp.where(better, bas[m], ba_sel)
            bf_bm = jnp.maximum(bf_bm, bms[m])
        idxs.append(base_sel + ba_sel)
        keeps.append(keep)
        txs.append(t1 * _IMG_SIZE)
        tys.append(t2 * _IMG_SIZE)
        tws.append(t3 * _IMG_SIZE)
        ths.append(t4 * _IMG_SIZE)
        wwhs.append(2.0 - t3 * t4)
        cis.append(t0.astype(jnp.int32))

    survs = []
    for n in range(_NB):
        s = keeps[n]
        for n2 in range(n + 1, _NB):
            s = s & jnp.logical_not(keeps[n2] & (idxs[n2] == idxs[n]))
        survs.append(s)

    lane = lax.broadcasted_iota(jnp.int32, (1, _WCOLS), 1)

    s_xywh = jnp.float32(0.0)
    s_conf1 = jnp.float32(0.0)
    s_cls = jnp.float32(0.0)
    cnt = jnp.float32(0.0)
    for n in range(_NB):
        sf = jnp.where(survs[n], 1.0, 0.0).astype(jnp.float32)
        rr = idxs[n] // _RPW
        off = (idxs[n] - rr * _RPW) * 85
        row = pred_ref[0, pl.ds(rr, 1), :]
        li = lane - off
        is_x = (li == 0).astype(jnp.float32)
        is_y = (li == 1).astype(jnp.float32)
        is_w = (li == 2).astype(jnp.float32)
        is_h = (li == 3).astype(jnp.float32)
        is_conf = (li == 4).astype(jnp.float32)
        is_xywh = ((li >= 0) & (li < 4)).astype(jnp.float32)
        is_cls = ((li >= 5) & (li < 85)).astype(jnp.float32)
        tvec = (txs[n] * is_x + tys[n] * is_y + tws[n] * is_w + ths[n] * is_h)
        d = row - tvec
        w2 = wwhs[n] * wwhs[n]
        s_xywh = s_xywh + sf * w2 * jnp.sum(d * d * is_xywh)
        pc = jnp.clip(row, 1e-7, 1.0 - 1e-7)
        lpc = jnp.log(pc)
        s_conf1 = s_conf1 + sf * jnp.sum(-lpc * is_conf)
        oh = (li == (5 + cis[n])).astype(jnp.float32)
        bce = -(oh * lpc + (1.0 - oh) * jnp.log(1.0 - pc))
        s_cls = s_cls + sf * jnp.sum(bce * is_cls)
        cnt = cnt + sf

    conf9 = jnp.concatenate(
        [pred_ref[0, :, 4 + 85 * k : 5 + 85 * k] for k in range(_RPW)],
        axis=1)
    nob9 = jnp.reshape(noobj_ref[:, :], (_WROWS, _RPW))
    u = jnp.clip(conf9 * nob9, 1e-7, 1.0 - 1e-7)
    f_dense = jnp.sum(-jnp.log(1.0 - u))

    acc_ref[0] = acc_ref[0] + s_xywh
    acc_ref[1] = acc_ref[1] + s_conf1
    acc_ref[2] = acc_ref[2] + s_cls
    acc_ref[3] = acc_ref[3] + cnt
    acc_ref[4] = acc_ref[4] + f_dense

    @pl.when(b == _BS - 1)
    def _():
        c0 = -jnp.log(1.0 - jnp.float32(1e-7))
        count = acc_ref[3]
        loss = (
            acc_ref[0] / _NTOT
            + (acc_ref[1] + (_NTOT - count) * c0 + 0.5 * acc_ref[4]) / _NTOT
            + acc_ref[2] / (count * _NUM_CLASSES)
        )
        out_ref[:, :] = jnp.full((1, 1), loss, jnp.float32)


@jax.jit
def _run(prediction, targets):
    pred2 = jnp.reshape(prediction, (_BS, _WROWS, _WCOLS))
    out = pl.pallas_call(
        _loss_body,
        grid=(_BS,),
        in_specs=[
            pl.BlockSpec((1, _WROWS, _WCOLS), lambda b: (b, 0, 0)),
            pl.BlockSpec((1, _NB, 5), lambda b: (b, 0, 0),
                         memory_space=pltpu.SMEM),
        ],
        out_specs=pl.BlockSpec((1, 1), lambda b: (0, 0)),
        out_shape=jax.ShapeDtypeStruct((1, 1), jnp.float32),
        scratch_shapes=[
            pltpu.VMEM((_TOTAL, 1), jnp.float32),
            pltpu.SMEM((8,), jnp.float32),
        ],
    )(pred2, targets)
    return out[0, 0]


def kernel(prediction, targets):
    return _run(prediction, targets)


# speedup vs baseline: 2.7128x; 2.7128x over previous
import functools
import numpy as np
import jax
import jax.numpy as jnp
from jax import lax
from jax.experimental import pallas as pl
from jax.experimental.pallas import tpu as pltpu

_NUM_CLASSES = 80
_IMG_SIZE = 416.0
_FM = [13, 26, 52]
_LAST = [0, 507, 2535]
_TOTAL = 10647
_ANCH = [
    [(3.625, 2.8125), (4.875, 6.1875), (11.65625, 10.1875)],
    [(1.875, 3.8125), (3.875, 2.8125), (3.6875, 7.4375)],
    [(1.25, 1.625), (2.0, 3.75), (4.125, 2.875)],
]
_BS = 32
_NB = 20
_NTOT = float(_BS * _TOTAL)


def _loss_body(pred_ref, tgt_ref, out_ref, noobj_ref, acc_ref):
    b = pl.program_id(0)

    @pl.when(b == 0)
    def _():
        for i in range(8):
            acc_ref[i] = 0.0

    noobj_ref[:, :] = jnp.ones((_TOTAL, 1), jnp.float32)

    idxs, keeps, txs, tys, tws, ths, wwhs, cis = [], [], [], [], [], [], [], []
    for n in range(_NB):
        t0 = tgt_ref[0, n, 0]
        t1 = tgt_ref[0, n, 1]
        t2 = tgt_ref[0, n, 2]
        t3 = tgt_ref[0, n, 3]
        t4 = tgt_ref[0, n, 4]
        keep = (t0 + t1 + t2 + t3 + t4) != 0.0

        bms, bas, bases = [], [], []
        for m in range(3):
            fs = float(_FM[m])
            gi = (t1 * fs).astype(jnp.int32)
            gj = (t2 * fs).astype(jnp.int32)
            gw = t3 * fs
            gh = t4 * fs
            ious = []
            for (aw, ah) in _ANCH[m]:
                inter = jnp.minimum(gw, aw) * jnp.minimum(gh, ah)
                union = gw * gh + aw * ah - inter + 1e-16
                ious.append(inter / union)
            bm = ious[0]
            ba = jnp.int32(0)
            ba = jnp.where(ious[1] > bm, 2 - 1, ba)
            bm = jnp.maximum(bm, ious[1])
            ba = jnp.where(ious[2] > bm, 2, ba)
            bm = jnp.maximum(bm, ious[2])
            base = _LAST[m] + 3 * gi * gj
            bases.append(base)
            bms.append(bm)
            bas.append(ba)
            for a in range(3):
                cond = keep & (ious[a] > 0.5)

                @pl.when(cond)
                def _(base=base, a=a):
                    noobj_ref[pl.ds(base + a, 1), :] = jnp.zeros((1, 1), jnp.float32)

        bf_bm = bms[0]
        base_sel = bases[0]
        ba_sel = bas[0]
        for m in (1, 2):
            better = bms[m] > bf_bm
            base_sel = jnp.where(better, bases[m], base_sel)
            ba_sel = jnp.where(better, bas[m], ba_sel)
            bf_bm = jnp.maximum(bf_bm, bms[m])
        idxs.append(base_sel + ba_sel)
        keeps.append(keep)
        txs.append(t1 * _IMG_SIZE)
        tys.append(t2 * _IMG_SIZE)
        tws.append(t3 * _IMG_SIZE)
        ths.append(t4 * _IMG_SIZE)
        wwhs.append(2.0 - t3 * t4)
        cis.append(t0.astype(jnp.int32))

    survs = []
    for n in range(_NB):
        s = keeps[n]
        for n2 in range(n + 1, _NB):
            s = s & jnp.logical_not(keeps[n2] & (idxs[n2] == idxs[n]))
        survs.append(s)

    lane = lax.broadcasted_iota(jnp.int32, (1, 85), 1)
    is_x = (lane == 0).astype(jnp.float32)
    is_y = (lane == 1).astype(jnp.float32)
    is_w = (lane == 2).astype(jnp.float32)
    is_h = (lane == 3).astype(jnp.float32)
    is_conf = (lane == 4).astype(jnp.float32)
    is_xywh = (lane < 4).astype(jnp.float32)
    is_cls = ((lane >= 5) & (lane < 85)).astype(jnp.float32)

    s_xywh = jnp.float32(0.0)
    s_conf1 = jnp.float32(0.0)
    s_cls = jnp.float32(0.0)
    cnt = jnp.float32(0.0)
    for n in range(_NB):
        sf = jnp.where(survs[n], 1.0, 0.0).astype(jnp.float32)
        row = pred_ref[0, pl.ds(idxs[n], 1), :]
        tvec = (txs[n] * is_x + tys[n] * is_y + tws[n] * is_w + ths[n] * is_h)
        d = row - tvec
        w2 = wwhs[n] * wwhs[n]
        s_xywh = s_xywh + sf * w2 * jnp.sum(d * d * is_xywh)
        pc = jnp.clip(row, 1e-7, 1.0 - 1e-7)
        s_conf1 = s_conf1 + sf * jnp.sum(-jnp.log(pc) * is_conf)
        oh = (lane == (5 + cis[n])).astype(jnp.float32)
        bce = -(oh * jnp.log(pc) + (1.0 - oh) * jnp.log(1.0 - pc))
        s_cls = s_cls + sf * jnp.sum(bce * is_cls)
        cnt = cnt + sf

    conf_col = pred_ref[0, :, 4:5]
    u = jnp.clip(conf_col * noobj_ref[:, :], 1e-7, 1.0 - 1e-7)
    f_dense = jnp.sum(-jnp.log(1.0 - u))

    acc_ref[0] = acc_ref[0] + s_xywh
    acc_ref[1] = acc_ref[1] + s_conf1
    acc_ref[2] = acc_ref[2] + s_cls
    acc_ref[3] = acc_ref[3] + cnt
    acc_ref[4] = acc_ref[4] + f_dense

    @pl.when(b == _BS - 1)
    def _():
        c0 = -jnp.log(1.0 - jnp.float32(1e-7))
        count = acc_ref[3]
        loss = (
            acc_ref[0] / _NTOT
            + (acc_ref[1] + (_NTOT - count) * c0 + 0.5 * acc_ref[4]) / _NTOT
            + acc_ref[2] / (count * _NUM_CLASSES)
        )
        out_ref[:, :] = jnp.full((1, 1), loss, jnp.float32)


@functools.partial(jax.jit, static_argnames=("interpret",))
def _run(prediction, targets, interpret=False):
    out = pl.pallas_call(
        _loss_body,
        grid=(_BS,),
        in_specs=[
            pl.BlockSpec((1, _TOTAL, 85), lambda b: (b, 0, 0)),
            pl.BlockSpec((1, _NB, 5), lambda b: (b, 0, 0),
                         memory_space=pltpu.SMEM),
        ],
        out_specs=pl.BlockSpec((1, 1), lambda b: (0, 0)),
        out_shape=jax.ShapeDtypeStruct((1, 1), jnp.float32),
        scratch_shapes=[
            pltpu.VMEM((_TOTAL, 1), jnp.float32),
            pltpu.SMEM((8,), jnp.float32),
        ],
        interpret=interpret,
    )(prediction, targets)
    return out[0, 0]


def kernel(prediction, targets):
    return _run(prediction, targets)
